# baseline (speedup 1.0000x reference)
import functools

import jax
import jax.numpy as jnp
from jax import lax
from jax.experimental import pallas as pl
from jax.experimental.pallas import tpu as pltpu
from jax.experimental.pallas import tpu_sc as plsc

BATCH = 4096
HIST = 20
DIM = 64
NUM_ROWS = BATCH * HIST
NPAIR = 500000
NC, NS = 2, 16
NW = NC * NS
RPW = NUM_ROWS // NW
CHUNK = 128
NCHUNK = RPW // CHUNK
L = 16
NTCOL = 7812
TAIL = 1000000 - NTCOL * 128
CPW = 245
TW = 133

_mesh = plsc.VectorSubcoreMesh(core_axis_name="c", subcore_axis_name="s")


def _iota():
    return lax.iota(jnp.int32, L)


def _splat(s):
    return jnp.full((L,), s, jnp.int32)


def _take16(v, idx):
    dnums = lax.GatherDimensionNumbers(
        offset_dims=(), collapsed_slice_dims=(0,), start_index_map=(0,)
    )
    return lax.gather(v, idx[:, None], dnums, slice_sizes=(1,),
                      mode=lax.GatherScatterMode.PROMISE_IN_BOUNDS)


@functools.partial(
    pl.kernel,
    mesh=_mesh,
    out_type=jax.ShapeDtypeStruct((NPAIR, 128), jnp.float32),
    scratch_types=[
        pltpu.VMEM((2, DIM, TW), jnp.float32),
        pltpu.VMEM((2, DIM, 128), jnp.float32),
        pltpu.SemaphoreType.DMA,
        pltpu.SemaphoreType.DMA,
        pltpu.SemaphoreType.DMA,
        pltpu.SemaphoreType.DMA,
    ],
    compiler_params=pltpu.CompilerParams(needs_layout_passes=False),
)
def _pair_transpose(tt_hbm, tailp_hbm, pairs_hbm, tbuf, obuf, sf0, sf1, sw0, sw1):
    wid = lax.axis_index("s") * NC + lax.axis_index("c")
    lo = wid * CPW
    hi = jnp.minimum(lo + CPW, NTCOL)
    sf = (sf0, sf1)
    sw = (sw0, sw1)

    def _fetch(col, b):
        pltpu.async_copy(
            tt_hbm.at[:, pl.ds(pl.multiple_of(col * 128, 128), 128)],
            tbuf.at[b, :, pl.ds(0, 128)],
            sf[b],
        )

    def _wait_fetch(b):
        pltpu.make_async_copy(
            tt_hbm.at[:, pl.ds(0, 128)], tbuf.at[b, :, pl.ds(0, 128)], sf[b]
        ).wait()

    def _wait_write(b):
        pltpu.make_async_copy(
            pairs_hbm.at[pl.ds(0, DIM)], obuf.at[b], sw[b]
        ).wait()

    def _do_transpose(b, npair_loc):
        def _row(p, _):
            c0 = _splat(2 * p)
            for q in range(8):
                rows = L * (q % 4) + _iota()
                cols = c0 + (q // 4)
                vals = plsc.load_gather(tbuf.at[b], [rows, cols])
                obuf[b, p, pl.ds(L * q, L)] = vals
            return 0

        lax.fori_loop(0, npair_loc, _row, 0)

    def _transpose(col, b, t):
        @pl.when(t > 0)
        def _():
            _wait_write(b)

        _do_transpose(b, DIM)
        pltpu.async_copy(
            obuf.at[b],
            pairs_hbm.at[pl.ds(pl.multiple_of(col * DIM, DIM), DIM)],
            sw[b],
        )

    @pl.when(lo < hi)
    def _prologue():
        _fetch(lo, 0)

    def _body(t, _):
        c0 = lo + 2 * t
        c1 = c0 + 1

        @pl.when(c1 < hi)
        def _f1():
            _fetch(c1, 1)

        @pl.when(c0 < hi)
        def _p0():
            _wait_fetch(0)
            _transpose(c0, 0, t)

        @pl.when(c1 + 1 < hi)
        def _f0():
            _fetch(c1 + 1, 0)

        @pl.when(c1 < hi)
        def _p1():
            _wait_fetch(1)
            _transpose(c1, 1, t)

        return 0

    lax.fori_loop(0, (CPW + 1) // 2, _body, 0)
    @pl.when(lo < hi)
    def _drain0():
        _wait_write(0)

    @pl.when(lo + 1 < hi)
    def _drain1():
        _wait_write(1)

    @pl.when(wid == 0)
    def _tail():
        pltpu.sync_copy(tailp_hbm, pairs_hbm.at[pl.ds(NTCOL * DIM, TAIL // 2)])


@functools.partial(
    pl.kernel,
    mesh=_mesh,
    out_type=jax.ShapeDtypeStruct((NUM_ROWS // 2, 128), jnp.float32),
    scratch_types=[
        pltpu.VMEM((RPW,), jnp.int32),
        pltpu.VMEM((RPW,), jnp.int32),
        pltpu.VMEM((RPW,), jnp.int32),
        pltpu.VMEM((2, CHUNK, 128), jnp.float32),
        pltpu.VMEM((2, CHUNK // 2, 128), jnp.float32),
        pltpu.SemaphoreType.DMA,
        pltpu.SemaphoreType.DMA,
        pltpu.SemaphoreType.DMA,
        pltpu.SemaphoreType.DMA,
    ],
    compiler_params=pltpu.CompilerParams(needs_layout_passes=False),
)
def _embed_gather(idx_hbm, pairs_hbm, out_hbm, idx_v, pidx_v, par_v,
                  pairbuf, outbuf, sg0, sg1, sw0, sw1):
    wid = lax.axis_index("s") * NC + lax.axis_index("c")
    base = wid * RPW
    pltpu.sync_copy(idx_hbm.at[pl.ds(pl.multiple_of(base, RPW), RPW)], idx_v)

    def _split(i, _):
        t = idx_v[pl.ds(i * L, L)]
        pidx_v[pl.ds(i * L, L)] = t >> 1
        par_v[pl.ds(i * L, L)] = (t & 1) << 6
        return 0

    lax.fori_loop(0, RPW // L, _split, 0)

    sg = (sg0, sg1)
    sw = (sw0, sw1)

    def _fire_gather(k):
        return pltpu.async_copy(
            pairs_hbm.at[pidx_v.at[pl.ds(k * CHUNK, CHUNK)]],
            pairbuf.at[k % 2],
            sg[k % 2],
        )

    gathers = [None] * NCHUNK
    writes = [None] * NCHUNK
    gathers[0] = _fire_gather(0)
    for k in range(NCHUNK):
        if k + 1 < NCHUNK:
            gathers[k + 1] = _fire_gather(k + 1)
        gathers[k].wait()
        if k >= 2:
            writes[k - 2].wait()

        def _select(r, _):
            g16 = (r >> 4) << 4
            par16 = par_v[pl.ds(k * CHUNK + g16, L)]
            par = _take16(par16, _splat(r & (L - 1)))
            for q in range(DIM // L):
                vals = plsc.load_gather(
                    pairbuf.at[k % 2], [_splat(r), par + (L * q + _iota())]
                )
                outbuf[k % 2, r >> 1, pl.ds(((r & 1) << 6) + L * q, L)] = vals
            return 0

        lax.fori_loop(0, CHUNK, _select, 0)
        writes[k] = pltpu.async_copy(
            outbuf.at[k % 2],
            out_hbm.at[pl.ds(wid * (RPW // 2) + k * (CHUNK // 2), CHUNK // 2)],
            sw[k % 2],
        )
    writes[NCHUNK - 2].wait()
    writes[NCHUNK - 1].wait()


def kernel(x, table):
    idx = x.reshape(NUM_ROWS).astype(jnp.int32)
    tailp = table[NTCOL * 128:].reshape(TAIL // 2, 128)
    pairs = _pair_transpose(table.T, tailp)
    out = _embed_gather(idx, pairs)
    return out.reshape(BATCH, HIST, DIM)


# speedup vs baseline: 2.4041x; 2.4041x over previous
import functools

import jax
import jax.numpy as jnp
from jax import lax
from jax.experimental import pallas as pl
from jax.experimental.pallas import tpu as pltpu
from jax.experimental.pallas import tpu_sc as plsc

BATCH = 4096
HIST = 20
DIM = 64
NUM_ROWS = BATCH * HIST
CHUNK = 128
N_CHUNKS = NUM_ROWS // CHUNK
NC, NS = 2, 16
NW = NC * NS
CHUNKS_PER_W = N_CHUNKS // NW
WAVE = CHUNKS_PER_W // 2

_mesh = plsc.VectorSubcoreMesh(core_axis_name="c", subcore_axis_name="s")


@functools.partial(
    pl.kernel,
    mesh=_mesh,
    out_type=jax.ShapeDtypeStruct((N_CHUNKS, CHUNK, DIM), jnp.float32),
    scratch_types=[
        pltpu.VMEM((CHUNKS_PER_W, CHUNK), jnp.int32),
        pltpu.VMEM((WAVE, CHUNK, DIM), jnp.float32),
        pltpu.SemaphoreType.DMA,
    ],
    compiler_params=pltpu.CompilerParams(use_tc_tiling_on_sc=False),
)
def _embed_gather(idx_hbm, table_hbm, out_hbm, idx_v, rows_v, sem):
    wid = lax.axis_index("s") * NC + lax.axis_index("c")
    base = wid * CHUNKS_PER_W
    pltpu.sync_copy(idx_hbm.at[wid], idx_v)
    for p in range(CHUNKS_PER_W // WAVE):
        copies = [
            pltpu.async_copy(
                table_hbm.at[idx_v.at[p * WAVE + j]], rows_v.at[j], sem
            )
            for j in range(WAVE)
        ]
        for c in copies:
            c.wait()
        pltpu.sync_copy(rows_v, out_hbm.at[pl.ds(base + p * WAVE, WAVE)])


def kernel(x, table):
    idx = x.reshape(NW, CHUNKS_PER_W, CHUNK).astype(jnp.int32)
    out = _embed_gather(idx, table)
    return out.reshape(BATCH, HIST, DIM)
